# baseline (speedup 1.0000x reference)
import jax
import jax.numpy as jnp
from jax import lax
from jax.experimental import pallas as pl
from jax.experimental.pallas import tpu as pltpu
from jax.experimental.pallas import tpu_sc as plsc

L = 16
NC = 2
NS = 16
NW = NC * NS
CHUNK = 128
NBUF = 2
SIZE_OUT = 128


def _sc_body(xf_hbm, w_hbm, no_hbm, out_hbm, w_v, no_v, xbufs, obufs,
             insem, outsem, wsem):
    size_in = w_hbm.shape[0]
    nvec_in = size_in // L
    batch = out_hbm.shape[0]
    rows_per_w = batch // NW
    nch = rows_per_w // CHUNK

    wid = lax.axis_index("s") * NC + lax.axis_index("c")
    base = wid * rows_per_w

    in_cp = [
        pltpu.make_async_copy(
            xf_hbm.at[pl.ds((base + ch * CHUNK) * size_in, CHUNK * size_in)],
            xbufs[ch % NBUF],
            insem.at[ch % NBUF])
        for ch in range(nch)
    ]
    out_cp = [
        pltpu.make_async_copy(
            obufs[ch % NBUF], out_hbm.at[pl.ds(base + ch * CHUNK, CHUNK)],
            outsem.at[ch % NBUF])
        for ch in range(nch)
    ]

    in_cp[0].start()
    w_cp = pltpu.make_async_copy(w_hbm, w_v, wsem)
    no_cp = pltpu.make_async_copy(no_hbm, no_v, wsem)
    w_cp.start()
    no_cp.start()

    zeros = jnp.zeros((L,), jnp.float32)
    for b in range(NBUF):
        @plsc.parallel_loop(0, CHUNK, unroll=4)
        def _(r, b=b):
            for k in range(SIZE_OUT // L):
                obufs[b][r, pl.ds(k * L, L)] = zeros

    w_cp.wait()
    no_cp.wait()
    wvecs = [w_v[pl.ds(j * L, L)] for j in range(nvec_in)]
    cvecs = [no_v[pl.ds(j * L, L)] for j in range(nvec_in)]

    for ch in range(nch):
        b = ch % NBUF
        if ch + 1 < nch:
            in_cp[ch + 1].start()
        in_cp[ch].wait()
        if ch >= NBUF:
            out_cp[ch - NBUF].wait()

        xoff = 0

        @plsc.parallel_loop(0, CHUNK, unroll=4)
        def _(r, b=b):
            ridx = jnp.full((L,), r, jnp.int32)
            for j in range(nvec_in):
                v = xbufs[b][pl.ds(r * size_in + j * L, L)] * wvecs[j]
                plsc.store_scatter(obufs[b], [ridx, cvecs[j]], v)

        out_cp[ch].start()
    for ch in range(max(0, nch - NBUF), nch):
        out_cp[ch].wait()


def _body(xf_hbm, w_hbm, no_hbm, out_hbm, w_v, no_v, xbuf0, xbuf1,
          obuf0, obuf1, insem, outsem, wsem):
    _sc_body(xf_hbm, w_hbm, no_hbm, out_hbm, w_v, no_v,
             (xbuf0, xbuf1), (obuf0, obuf1), insem, outsem, wsem)


@jax.jit
def kernel(x, weights, node_order):
    B, size_in = x.shape
    mesh = plsc.VectorSubcoreMesh(core_axis_name="c", subcore_axis_name="s")
    f = pl.kernel(
        _body,
        out_type=jax.ShapeDtypeStruct((B, SIZE_OUT), x.dtype),
        mesh=mesh,
        compiler_params=pltpu.CompilerParams(needs_layout_passes=False),
        scratch_types=[
            pltpu.VMEM((size_in,), jnp.float32),
            pltpu.VMEM((size_in,), jnp.int32),
            pltpu.VMEM((CHUNK * size_in,), jnp.float32),
            pltpu.VMEM((CHUNK * size_in,), jnp.float32),
            pltpu.VMEM((CHUNK, SIZE_OUT), jnp.float32),
            pltpu.VMEM((CHUNK, SIZE_OUT), jnp.float32),
            pltpu.SemaphoreType.DMA((NBUF,)),
            pltpu.SemaphoreType.DMA((NBUF,)),
            pltpu.SemaphoreType.DMA,
        ],
    )
    return f(x.reshape(B * size_in), weights, node_order)


# speedup vs baseline: 1.2377x; 1.2377x over previous
import jax
import jax.numpy as jnp
from jax import lax
from jax.experimental import pallas as pl
from jax.experimental.pallas import tpu as pltpu
from jax.experimental.pallas import tpu_sc as plsc

L = 16
NC = 2
NS = 16
NW = NC * NS
CHUNK = 128
NBUF = 2
SIZE_OUT = 128


def _sc_body(x_hbm, w_hbm, no_hbm, out_hbm, w_v, no_v, xbufs, obufs,
             insem, outsem, wsem):
    size_in = x_hbm.shape[1]
    nvec_in = size_in // L
    rows_per_w = x_hbm.shape[0] // NW
    nch = rows_per_w // CHUNK
    nsuper = nch // NBUF

    wid = lax.axis_index("s") * NC + lax.axis_index("c")
    base = wid * rows_per_w

    def start_in(ch, b):
        pltpu.make_async_copy(
            x_hbm.at[pl.ds(base + ch * CHUNK, CHUNK)], xbufs[b],
            insem.at[b]).start()

    def wait_in(b):
        pltpu.make_async_copy(
            x_hbm.at[pl.ds(0, CHUNK)], xbufs[b], insem.at[b]).wait()

    def start_out(ch, b):
        pltpu.make_async_copy(
            obufs[b], out_hbm.at[pl.ds(base + ch * CHUNK, CHUNK)],
            outsem.at[b]).start()

    def wait_out(b):
        pltpu.make_async_copy(
            obufs[b], out_hbm.at[pl.ds(0, CHUNK)], outsem.at[b]).wait()

    start_in(0, 0)
    start_in(1, 1)
    w_cp = pltpu.make_async_copy(w_hbm, w_v, wsem)
    no_cp = pltpu.make_async_copy(no_hbm, no_v, wsem)
    w_cp.start()
    no_cp.start()

    zeros = jnp.zeros((L,), jnp.float32)
    for b in range(NBUF):
        @plsc.parallel_loop(0, CHUNK, unroll=4)
        def _(r, b=b):
            for k in range(SIZE_OUT // L):
                obufs[b][r, pl.ds(k * L, L)] = zeros

    w_cp.wait()
    no_cp.wait()
    wvecs = [w_v[pl.ds(j * L, L)] for j in range(nvec_in)]
    cvecs = [no_v[pl.ds(j * L, L)] for j in range(nvec_in)]

    def compute(b):
        @plsc.parallel_loop(0, CHUNK, unroll=4)
        def _(r, b=b):
            ridx = jnp.full((L,), r, jnp.int32)
            for j in range(nvec_in):
                v = xbufs[b][r, pl.ds(j * L, L)] * wvecs[j]
                plsc.store_scatter(obufs[b], [ridx, cvecs[j]], v)

    for b in range(NBUF):
        wait_in(b)
        compute(b)
        start_in(NBUF + b, b)
        start_out(b, b)

    @pl.loop(1, nsuper - 1)
    def _(g):
        ch0 = g * NBUF
        for b in range(NBUF):
            wait_out(b)
            wait_in(b)
            compute(b)
            start_in(ch0 + NBUF + b, b)
            start_out(ch0 + b, b)

    ch0 = (nsuper - 1) * NBUF
    for b in range(NBUF):
        wait_out(b)
        wait_in(b)
        compute(b)
        start_out(ch0 + b, b)
    for b in range(NBUF):
        wait_out(b)


def _body(x_hbm, w_hbm, no_hbm, out_hbm, w_v, no_v, xbuf0, xbuf1,
          obuf0, obuf1, insem, outsem, wsem):
    _sc_body(x_hbm, w_hbm, no_hbm, out_hbm, w_v, no_v,
             (xbuf0, xbuf1), (obuf0, obuf1), insem, outsem, wsem)


@jax.jit
def kernel(x, weights, node_order):
    B, size_in = x.shape
    mesh = plsc.VectorSubcoreMesh(core_axis_name="c", subcore_axis_name="s")
    f = pl.kernel(
        _body,
        out_type=jax.ShapeDtypeStruct((B, SIZE_OUT), x.dtype),
        mesh=mesh,
        compiler_params=pltpu.CompilerParams(needs_layout_passes=False),
        scratch_types=[
            pltpu.VMEM((size_in,), jnp.float32),
            pltpu.VMEM((size_in,), jnp.int32),
            pltpu.VMEM((CHUNK, size_in), jnp.float32),
            pltpu.VMEM((CHUNK, size_in), jnp.float32),
            pltpu.VMEM((CHUNK, SIZE_OUT), jnp.float32),
            pltpu.VMEM((CHUNK, SIZE_OUT), jnp.float32),
            pltpu.SemaphoreType.DMA((NBUF,)),
            pltpu.SemaphoreType.DMA((NBUF,)),
            pltpu.SemaphoreType.DMA,
        ],
    )
    return f(x, weights, node_order)
